# baseline (speedup 1.0000x reference)
import functools

import jax
import jax.numpy as jnp
from jax import lax
from jax.experimental import pallas as pl
from jax.experimental.pallas import tpu as pltpu
import jax.experimental.pallas.tpu_sc as plsc

N = 10000
E = 320000
DIN = 128
HID = 32
P = 12
F = 3 * HID
ER = E // 128
NT = 16
NPAD = 16384
NP2 = 10240
FP = 128
ERP = 2688
RPT = ERP // NT

_mesh = plsc.VectorSubcoreMesh(core_axis_name="c", subcore_axis_name="s")
_sc_params = pltpu.CompilerParams(needs_layout_passes=False)


@functools.partial(
    pl.kernel,
    out_type=jax.ShapeDtypeStruct((2 * NPAD,), jnp.float32),
    mesh=_mesh,
    compiler_params=_sc_params,
    scratch_types=[
        pltpu.VMEM((1, 128), jnp.int32),
        pltpu.VMEM((1, 128), jnp.float32),
        pltpu.VMEM_SHARED((NPAD,), jnp.float32),
    ],
)
def _deg_kernel(dst_hbm, w_hbm, zeros_hbm, out_hbm, idx_v, w_v, deg_sp):
    c = lax.axis_index("c")
    s = lax.axis_index("s")
    w = c * NT + s

    pltpu.sync_copy(zeros_hbm.at[pl.ds(s * 1024, 1024)],
                    deg_sp.at[pl.ds(s * 1024, 1024)])

    plsc.subcore_barrier()

    trips = (ER - w + 31) // 32

    def body(k, carry):
        r = w + k * 32
        pltpu.sync_copy(dst_hbm.at[pl.ds(r, 1)], idx_v)
        pltpu.sync_copy(w_hbm.at[pl.ds(r, 1)], w_v)
        pltpu.sync_copy(w_v.at[0], deg_sp.at[idx_v.at[0]], add=True)
        return carry

    lax.fori_loop(0, trips, body, 0)
    plsc.subcore_barrier()

    pltpu.sync_copy(deg_sp.at[pl.ds(s * 1024, 1024)],
                    out_hbm.at[pl.ds(c * NPAD + s * 1024, 1024)])


@functools.partial(
    pl.kernel,
    out_type=jax.ShapeDtypeStruct((P, NP2, FP), jnp.float32),
    mesh=_mesh,
    compiler_params=_sc_params,
    scratch_types=[
        pltpu.VMEM((128, FP), jnp.float32),
        pltpu.VMEM((128, FP), jnp.float32),
        pltpu.VMEM((1, 2, 128), jnp.int32),
        pltpu.VMEM((1, 2, 128), jnp.int32),
        pltpu.VMEM((1, 2, 128), jnp.int32),
        pltpu.VMEM((256,), jnp.float32),
        pltpu.VMEM_SHARED((NP2, FP), jnp.float32),
        pltpu.SemaphoreType.DMA,
        pltpu.SemaphoreType.DMA,
        pltpu.SemaphoreType.DMA,
        pltpu.SemaphoreType.DMA,
        pltpu.SemaphoreType.DMA,
        pltpu.SemaphoreType.DMA,
        pltpu.SemaphoreType.DMA,
    ],
)
def _prop_kernel(packed_hbm, g_hbm, out_hbm, ba, bb, i0, i1, i2, wfl,
                 u_sp, sga, sgb, ssa, ssb, se0, se1, se2):
    c = lax.axis_index("c")
    s = lax.axis_index("s")
    bufs = (ba, bb)
    gsems = (sga, sgb)
    ssems = (ssa, ssb)
    ibufs = (i0, i1, i2)
    esems = (se0, se1, se2)

    def e_desc(r, j):
        return pltpu.make_async_copy(
            packed_hbm.at[pl.ds(s * RPT + r, 1)], ibufs[j], esems[j])

    def unpack(j, x):
        ib = ibufs[j]
        for k in range(8):
            v = ib[0, 0, pl.ds(k * 16, 16)]
            wb = ib[0, 1, pl.ds(k * 16, 16)]
            wfl[pl.ds(x * 128 + k * 16, 16)] = plsc.bitcast(wb, jnp.float32)
            ib[0, 0, pl.ds(k * 16, 16)] = v & 0xFFFF
            ib[0, 1, pl.ds(k * 16, 16)] = lax.shift_right_logical(v, 16)

    def scale(x):
        buf = bufs[x]
        def sbody(e, cc):
            wv = plsc.load_gather(
                wfl, [jnp.full((16,), x * 128, jnp.int32) + e])
            for j in range(F // 16):
                buf[e, pl.ds(j * 16, 16)] = buf[e, pl.ds(j * 16, 16)] * wv
            return cc
        return sbody

    def zinit(i, carry):
        for j in range(FP // 16):
            ba[i, pl.ds(j * 16, 16)] = jnp.zeros((16,), jnp.float32)
        return carry

    def period(t, carry):
        p = c * (P // 2) + t

        lax.fori_loop(0, 128, zinit, 0)
        for q in range(5):
            pltpu.sync_copy(ba, u_sp.at[pl.ds(s * 640 + q * 128, 128)])
        plsc.subcore_barrier()

        def gath(j, x):
            return pltpu.make_async_copy(
                g_hbm.at[p].at[ibufs[j].at[0].at[0]], bufs[x], gsems[x])

        def scat(j, x):
            return pltpu.make_async_copy(
                bufs[x], u_sp.at[ibufs[j].at[0].at[1]], ssems[x])

        e_desc(0, 0).start()
        e_desc(1, 1).start()
        e_desc(0, 0).wait()
        unpack(0, 0)
        gath(0, 0).start()

        def step(r, x, i):
            ni = (i + 1) % 3
            fi = (i + 2) % 3
            y = 1 - x
            gath(i, x).wait()
            lax.fori_loop(0, 64, scale(x), 0, unroll=8)

            @pl.when(r >= 1)
            def _():
                scat(fi, y).wait()

            @pl.when(r + 2 < RPT)
            def _():
                e_desc(r + 2, fi).start()

            @pl.when(r + 1 < RPT)
            def _():
                e_desc(r + 1, ni).wait()
                unpack(ni, y)
                gath(ni, y).start()

            lax.fori_loop(64, 128, scale(x), 0, unroll=8)
            pltpu.async_copy(bufs[x], u_sp.at[ibufs[i].at[0].at[1]], ssems[x],
                             add=True)

        def body(kk, cc):
            r = kk * 6
            for bq in range(6):
                step(r + bq, bq % 2, bq % 3)
            return cc

        lax.fori_loop(0, RPT // 6, body, 0)
        scat((RPT - 1) % 3, (RPT - 1) % 2).wait()
        plsc.subcore_barrier()
        for q in range(5):
            off = s * 640 + q * 128
            pltpu.sync_copy(u_sp.at[pl.ds(off, 128)],
                            out_hbm.at[p].at[pl.ds(off, 128)])
        plsc.subcore_barrier()
        return carry

    lax.fori_loop(0, P // 2, period, 0)


def _dinv_body(deg_ref, o_ref):
    o_ref[...] = lax.rsqrt(deg_ref[0:1, :] + deg_ref[1:2, :] + 1.0)


def _mm_body(xt_ref, w_ref, dinv_ref, o_ref):
    g = jnp.dot(xt_ref[0], w_ref[...], preferred_element_type=jnp.float32)
    o_ref[0] = g * dinv_ref[...]


def _gru_body(scat_ref, g_ref, dinv_ref, b_ref, wlzb_ref, wlrb_ref, wlhb_ref,
              probs_ref, wout_ref, bout_ref, o_ref):
    bn = scat_ref.shape[1]
    u = (scat_ref[...] + g_ref[...]) * dinv_ref[...][None]
    u = u + b_ref[...][None]
    h = jnp.zeros((bn, HID), jnp.float32)
    hacc = jnp.zeros((bn, HID), jnp.float32)
    wlzb = wlzb_ref[...]
    wlrb = wlrb_ref[...]
    wlhb = wlhb_ref[...]
    for t in range(P):
        ut = u[t]
        z = jax.nn.sigmoid(ut[:, :HID] + jnp.dot(h, wlzb, preferred_element_type=jnp.float32))
        r = jax.nn.sigmoid(ut[:, HID:2 * HID] + jnp.dot(h, wlrb, preferred_element_type=jnp.float32))
        ht = jnp.tanh(ut[:, 2 * HID:3 * HID] + jnp.dot(h * r, wlhb, preferred_element_type=jnp.float32))
        h = z * h + (1.0 - z) * ht
        hacc = hacc + probs_ref[0, t] * h
    o_ref[...] = jnp.dot(jnp.maximum(hacc, 0.0), wout_ref[...],
                         preferred_element_type=jnp.float32) + bout_ref[...]


def kernel(x, edge_index, edge_attr, Wz, bz, Wlz, blz, Wr, br, Wlr, blr,
           Wh, bh, Wlh, blh, att, Wout, bout):
    f32 = jnp.float32
    Wcat = jnp.concatenate(
        [Wz @ Wlz[:HID], Wr @ Wlr[:HID], Wh @ Wlh[:HID]], axis=1)
    b96 = jnp.concatenate(
        [bz @ Wlz[:HID] + blz, br @ Wlr[:HID] + blr, bh @ Wlh[:HID] + blh])
    probs = jax.nn.softmax(att)
    xt = jnp.transpose(x, (2, 0, 1))
    ei3 = edge_index.reshape(2, ER, 128)
    src2, dst2 = ei3[0], ei3[1]
    w2 = edge_attr.reshape(ER, 128)
    zeros_n = jnp.zeros((NPAD,), f32)

    degp = _deg_kernel(dst2, w2, zeros_n)
    dinv = pl.pallas_call(
        _dinv_body,
        out_shape=jax.ShapeDtypeStruct((1, NPAD), f32),
    )(degp.reshape(2, NPAD))
    dinv_col = dinv[0, :N].reshape(N, 1)

    BN = 400
    g = pl.pallas_call(
        _mm_body,
        grid=(P, N // BN),
        in_specs=[
            pl.BlockSpec((1, BN, DIN), lambda t, i: (t, i, 0)),
            pl.BlockSpec((DIN, FP), lambda t, i: (0, 0)),
            pl.BlockSpec((BN, 1), lambda t, i: (i, 0)),
        ],
        out_specs=pl.BlockSpec((1, BN, FP), lambda t, i: (t, i, 0)),
        out_shape=jax.ShapeDtypeStruct((P, NP2, FP), f32),
    )(xt, jnp.pad(Wcat, ((0, 0), (0, FP - F))), dinv_col)

    npad_e = ERP * 128 - E
    pad_idx = (jnp.arange(npad_e, dtype=jnp.int32) * 97) % N
    src_p = jnp.concatenate([edge_index[0], pad_idx])
    dst_p = jnp.concatenate([edge_index[1], pad_idx])
    w_p = jnp.concatenate([edge_attr, jnp.zeros((npad_e,), f32)])
    packed = jnp.concatenate(
        [(src_p + dst_p * 65536).reshape(ERP, 1, 128),
         jax.lax.bitcast_convert_type(w_p, jnp.int32).reshape(ERP, 1, 128)],
        axis=1)
    scat = _prop_kernel(packed, g)

    out = pl.pallas_call(
        _gru_body,
        grid=(N // BN,),
        in_specs=[
            pl.BlockSpec((P, BN, FP), lambda i: (0, i, 0)),
            pl.BlockSpec((P, BN, FP), lambda i: (0, i, 0)),
            pl.BlockSpec((BN, 1), lambda i: (i, 0)),
            pl.BlockSpec((1, FP), lambda i: (0, 0)),
            pl.BlockSpec((HID, HID), lambda i: (0, 0)),
            pl.BlockSpec((HID, HID), lambda i: (0, 0)),
            pl.BlockSpec((HID, HID), lambda i: (0, 0)),
            pl.BlockSpec((1, P), lambda i: (0, 0)),
            pl.BlockSpec((HID, P), lambda i: (0, 0)),
            pl.BlockSpec((1, P), lambda i: (0, 0)),
        ],
        out_specs=pl.BlockSpec((BN, P), lambda i: (i, 0)),
        out_shape=jax.ShapeDtypeStruct((N, P), f32),
    )(scat, g, dinv_col, jnp.pad(b96, (0, FP - F)).reshape(1, FP), Wlz[HID:], Wlr[HID:], Wlh[HID:],
      probs.reshape(1, P), Wout, bout.reshape(1, P))
    return out


# speedup vs baseline: 1.2743x; 1.2743x over previous
import functools

import jax
import jax.numpy as jnp
from jax import lax
from jax.experimental import pallas as pl
from jax.experimental.pallas import tpu as pltpu
import jax.experimental.pallas.tpu_sc as plsc

N = 10000
E = 320000
DIN = 128
HID = 32
P = 12
F = 3 * HID
ER = E // 128
NT = 16
NPAD = 16384
NP2 = 10048
FP = 128
ERP = 2688
RPT = ERP // NT

_mesh = plsc.VectorSubcoreMesh(core_axis_name="c", subcore_axis_name="s")
_sc_params = pltpu.CompilerParams(needs_layout_passes=False)


@functools.partial(
    pl.kernel,
    out_type=jax.ShapeDtypeStruct((2 * NPAD,), jnp.float32),
    mesh=_mesh,
    compiler_params=_sc_params,
    scratch_types=[
        pltpu.VMEM((1, 128), jnp.int32),
        pltpu.VMEM((1, 128), jnp.float32),
        pltpu.VMEM_SHARED((NPAD,), jnp.float32),
    ],
)
def _deg_kernel(dst_hbm, w_hbm, zeros_hbm, out_hbm, idx_v, w_v, deg_sp):
    c = lax.axis_index("c")
    s = lax.axis_index("s")
    w = c * NT + s

    pltpu.sync_copy(zeros_hbm.at[pl.ds(s * 1024, 1024)],
                    deg_sp.at[pl.ds(s * 1024, 1024)])

    plsc.subcore_barrier()

    trips = (ER - w + 31) // 32

    def body(k, carry):
        r = w + k * 32
        pltpu.sync_copy(dst_hbm.at[pl.ds(r, 1)], idx_v)
        pltpu.sync_copy(w_hbm.at[pl.ds(r, 1)], w_v)
        pltpu.sync_copy(w_v.at[0], deg_sp.at[idx_v.at[0]], add=True)
        return carry

    lax.fori_loop(0, trips, body, 0)
    plsc.subcore_barrier()

    pltpu.sync_copy(deg_sp.at[pl.ds(s * 1024, 1024)],
                    out_hbm.at[pl.ds(c * NPAD + s * 1024, 1024)])


@functools.partial(
    pl.kernel,
    out_type=jax.ShapeDtypeStruct((P, NP2, FP), jnp.float32),
    mesh=_mesh,
    compiler_params=_sc_params,
    scratch_types=[
        pltpu.VMEM((128, FP), jnp.float32),
        pltpu.VMEM((128, FP), jnp.float32),
        pltpu.VMEM((128, FP), jnp.float32),
        pltpu.VMEM((1, 2, 128), jnp.int32),
        pltpu.VMEM((1, 2, 128), jnp.int32),
        pltpu.VMEM((1, 2, 128), jnp.int32),
        pltpu.VMEM((1, 2, 128), jnp.int32),
        pltpu.VMEM((384,), jnp.float32),
        pltpu.VMEM_SHARED((NP2, FP), jnp.float32),
        pltpu.SemaphoreType.DMA,
        pltpu.SemaphoreType.DMA,
        pltpu.SemaphoreType.DMA,
        pltpu.SemaphoreType.DMA,
        pltpu.SemaphoreType.DMA,
        pltpu.SemaphoreType.DMA,
        pltpu.SemaphoreType.DMA,
        pltpu.SemaphoreType.DMA,
        pltpu.SemaphoreType.DMA,
        pltpu.SemaphoreType.DMA,
    ],
)
def _prop_kernel(packed_hbm, g_hbm, out_hbm, b0, b1, b2, i0, i1, i2, i3, wfl,
                 u_sp, sg0, sg1, sg2, ss0, ss1, ss2, se0, se1, se2, se3):
    c = lax.axis_index("c")
    s = lax.axis_index("s")
    bufs = (b0, b1, b2)
    gsems = (sg0, sg1, sg2)
    ssems = (ss0, ss1, ss2)
    ibufs = (i0, i1, i2, i3)
    esems = (se0, se1, se2, se3)

    def e_desc(r, m):
        return pltpu.make_async_copy(
            packed_hbm.at[pl.ds(s * RPT + r, 1)], ibufs[m], esems[m])

    def unpack(m, q):
        ib = ibufs[m]
        for k in range(8):
            v = ib[0, 0, pl.ds(k * 16, 16)]
            wb = ib[0, 1, pl.ds(k * 16, 16)]
            wfl[pl.ds(q * 128 + k * 16, 16)] = plsc.bitcast(wb, jnp.float32)
            ib[0, 0, pl.ds(k * 16, 16)] = v & 0xFFFF
            ib[0, 1, pl.ds(k * 16, 16)] = lax.shift_right_logical(v, 16)

    def scale(q):
        buf = bufs[q]
        def sbody(e, cc):
            wv = plsc.load_gather(
                wfl, [jnp.full((16,), q * 128, jnp.int32) + e])
            for j in range(F // 16):
                buf[e, pl.ds(j * 16, 16)] = buf[e, pl.ds(j * 16, 16)] * wv
            return cc
        return sbody

    def zinit(i, carry):
        for j in range(FP // 16):
            b0[i, pl.ds(j * 16, 16)] = jnp.zeros((16,), jnp.float32)
        return carry

    zoff = jnp.minimum(s * 632, NP2 - 632)

    def period(t, carry):
        p = c * (P // 2) + t

        lax.fori_loop(0, 128, zinit, 0)
        for q in range(4):
            pltpu.sync_copy(b0, u_sp.at[pl.ds(zoff + q * 128, 128)])
        pltpu.sync_copy(b0.at[pl.ds(0, 120)],
                        u_sp.at[pl.ds(zoff + 512, 120)])
        plsc.subcore_barrier()

        def gath(m, q):
            return pltpu.make_async_copy(
                g_hbm.at[p].at[ibufs[m].at[0].at[0]], bufs[q], gsems[q])

        def scat(m, q):
            return pltpu.make_async_copy(
                bufs[q], u_sp.at[ibufs[m].at[0].at[1]], ssems[q])

        e_desc(0, 0).start()
        e_desc(1, 1).start()
        e_desc(2, 2).start()
        e_desc(0, 0).wait()
        unpack(0, 0)
        gath(0, 0).start()
        e_desc(1, 1).wait()
        unpack(1, 1)
        gath(1, 1).start()

        def step(r, q, m):
            gath(m, q).wait()
            lax.fori_loop(0, 128, scale(q), 0, unroll=8)

            @pl.when(r >= 1)
            def _():
                scat((m + 3) % 4, (q + 2) % 3).wait()

            @pl.when(r + 3 < RPT)
            def _():
                e_desc(r + 3, (m + 3) % 4).start()

            @pl.when(r + 2 < RPT)
            def _():
                e_desc(r + 2, (m + 2) % 4).wait()
                unpack((m + 2) % 4, (q + 2) % 3)
                gath((m + 2) % 4, (q + 2) % 3).start()

            pltpu.async_copy(bufs[q], u_sp.at[ibufs[m].at[0].at[1]], ssems[q],
                             add=True)

        def body(kk, cc):
            r = kk * 12
            for bq in range(12):
                step(r + bq, bq % 3, bq % 4)
            return cc

        lax.fori_loop(0, RPT // 12, body, 0)
        scat((RPT - 1) % 4, (RPT - 1) % 3).wait()
        plsc.subcore_barrier()
        for q in range(4):
            pltpu.sync_copy(u_sp.at[pl.ds(zoff + q * 128, 128)],
                            out_hbm.at[p].at[pl.ds(zoff + q * 128, 128)])
        pltpu.sync_copy(u_sp.at[pl.ds(zoff + 512, 120)],
                        out_hbm.at[p].at[pl.ds(zoff + 512, 120)])
        plsc.subcore_barrier()
        return carry

    lax.fori_loop(0, P // 2, period, 0)


def _dinv_body(deg_ref, o_ref):
    o_ref[...] = lax.rsqrt(deg_ref[0:1, :] + deg_ref[1:2, :] + 1.0)


def _mm_body(xt_ref, w_ref, dinv_ref, o_ref):
    g = jnp.dot(xt_ref[0], w_ref[...], preferred_element_type=jnp.float32)
    o_ref[0] = g * dinv_ref[...]


def _gru_body(scat_ref, g_ref, dinv_ref, b_ref, wlzb_ref, wlrb_ref, wlhb_ref,
              probs_ref, wout_ref, bout_ref, o_ref):
    bn = scat_ref.shape[1]
    u = (scat_ref[...] + g_ref[...]) * dinv_ref[...][None]
    u = u + b_ref[...][None]
    h = jnp.zeros((bn, HID), jnp.float32)
    hacc = jnp.zeros((bn, HID), jnp.float32)
    wlzb = wlzb_ref[...]
    wlrb = wlrb_ref[...]
    wlhb = wlhb_ref[...]
    for t in range(P):
        ut = u[t]
        z = jax.nn.sigmoid(ut[:, :HID] + jnp.dot(h, wlzb, preferred_element_type=jnp.float32))
        r = jax.nn.sigmoid(ut[:, HID:2 * HID] + jnp.dot(h, wlrb, preferred_element_type=jnp.float32))
        ht = jnp.tanh(ut[:, 2 * HID:3 * HID] + jnp.dot(h * r, wlhb, preferred_element_type=jnp.float32))
        h = z * h + (1.0 - z) * ht
        hacc = hacc + probs_ref[0, t] * h
    o_ref[...] = jnp.dot(jnp.maximum(hacc, 0.0), wout_ref[...],
                         preferred_element_type=jnp.float32) + bout_ref[...]


def kernel(x, edge_index, edge_attr, Wz, bz, Wlz, blz, Wr, br, Wlr, blr,
           Wh, bh, Wlh, blh, att, Wout, bout):
    f32 = jnp.float32
    Wcat = jnp.concatenate(
        [Wz @ Wlz[:HID], Wr @ Wlr[:HID], Wh @ Wlh[:HID]], axis=1)
    b96 = jnp.concatenate(
        [bz @ Wlz[:HID] + blz, br @ Wlr[:HID] + blr, bh @ Wlh[:HID] + blh])
    probs = jax.nn.softmax(att)
    xt = jnp.transpose(x, (2, 0, 1))
    ei3 = edge_index.reshape(2, ER, 128)
    src2, dst2 = ei3[0], ei3[1]
    w2 = edge_attr.reshape(ER, 128)
    zeros_n = jnp.zeros((NPAD,), f32)

    degp = _deg_kernel(dst2, w2, zeros_n)
    dinv = pl.pallas_call(
        _dinv_body,
        out_shape=jax.ShapeDtypeStruct((1, NPAD), f32),
    )(degp.reshape(2, NPAD))
    dinv_col = dinv[0, :N].reshape(N, 1)

    BN = 400
    g = pl.pallas_call(
        _mm_body,
        grid=(P, N // BN),
        in_specs=[
            pl.BlockSpec((1, BN, DIN), lambda t, i: (t, i, 0)),
            pl.BlockSpec((DIN, FP), lambda t, i: (0, 0)),
            pl.BlockSpec((BN, 1), lambda t, i: (i, 0)),
        ],
        out_specs=pl.BlockSpec((1, BN, FP), lambda t, i: (t, i, 0)),
        out_shape=jax.ShapeDtypeStruct((P, NP2, FP), f32),
    )(xt, jnp.pad(Wcat, ((0, 0), (0, FP - F))), dinv_col)

    npad_e = ERP * 128 - E
    pad_idx = (jnp.arange(npad_e, dtype=jnp.int32) * 97) % N
    src_p = jnp.concatenate([edge_index[0], pad_idx])
    dst_p = jnp.concatenate([edge_index[1], pad_idx])
    w_p = jnp.concatenate([edge_attr, jnp.zeros((npad_e,), f32)])
    packed = jnp.concatenate(
        [(src_p + dst_p * 65536).reshape(ERP, 1, 128),
         jax.lax.bitcast_convert_type(w_p, jnp.int32).reshape(ERP, 1, 128)],
        axis=1)
    scat = _prop_kernel(packed, g)

    out = pl.pallas_call(
        _gru_body,
        grid=(N // BN,),
        in_specs=[
            pl.BlockSpec((P, BN, FP), lambda i: (0, i, 0)),
            pl.BlockSpec((P, BN, FP), lambda i: (0, i, 0)),
            pl.BlockSpec((BN, 1), lambda i: (i, 0)),
            pl.BlockSpec((1, FP), lambda i: (0, 0)),
            pl.BlockSpec((HID, HID), lambda i: (0, 0)),
            pl.BlockSpec((HID, HID), lambda i: (0, 0)),
            pl.BlockSpec((HID, HID), lambda i: (0, 0)),
            pl.BlockSpec((1, P), lambda i: (0, 0)),
            pl.BlockSpec((HID, P), lambda i: (0, 0)),
            pl.BlockSpec((1, P), lambda i: (0, 0)),
        ],
        out_specs=pl.BlockSpec((BN, P), lambda i: (i, 0)),
        out_shape=jax.ShapeDtypeStruct((N, P), f32),
    )(scat, g, dinv_col, jnp.pad(b96, (0, FP - F)).reshape(1, FP), Wlz[HID:], Wlr[HID:], Wlh[HID:],
      probs.reshape(1, P), Wout, bout.reshape(1, P))
    return out
